# Initial kernel scaffold; baseline (speedup 1.0000x reference)
import jax
import jax.numpy as jnp
from jax.experimental import pallas as pl


def kernel(src_ids, pos_dst_ids, neg_dst_ids, node_feat, edge_index, Wn0, Wr0, b0, Wn1, Wr1, b1, w_pred):
    raise NotImplementedError("write your pallas kernel here")


# speedup vs baseline: 6.3991x; 6.3991x over previous
import jax
import jax.numpy as jnp
from jax import lax
from jax.experimental import pallas as pl
from jax.experimental.pallas import tpu as pltpu
from jax.experimental.pallas import tpu_sc as plsc

N = 10000
NPAD = 10240
E = 320000
D_IN = 128
D_H = 256
D_OUT = 128
BATCH = 8192
IDS = 3 * BATCH

NC, NS = 2, 16
NW = NC * NS
EPW = E // NW
C = 80
NCH = EPW // C
RPT = NPAD // NS
IPW = IDS // NW
GC = 128

_mesh = plsc.VectorSubcoreMesh(
    core_axis_name="c", subcore_axis_name="s", num_cores=NC, num_subcores=NS)


def _make_edge_agg(with_deg):
  out_types = [jax.ShapeDtypeStruct((NC * NPAD, 128), jnp.float32)]
  scratch = [
      pltpu.VMEM((C,), jnp.int32),
      pltpu.VMEM((C,), jnp.int32),
      pltpu.VMEM((C, 128), jnp.float32),
      pltpu.VMEM((C, 128), jnp.float32),
      pltpu.VMEM_SHARED((NPAD, 128), jnp.float32),
      pltpu.SemaphoreType.DMA,
  ]
  if with_deg:
    out_types.append(jax.ShapeDtypeStruct((NC * NPAD, 16), jnp.float32))
    scratch += [
        pltpu.VMEM((C, 16), jnp.float32),
        pltpu.VMEM((RPT, 16), jnp.float32),
        pltpu.VMEM_SHARED((NPAD, 16), jnp.float32),
    ]

  def body(x_hbm, src_hbm, dst_hbm, *refs):
    if with_deg:
      (agg_out, deg_out, src_v, dst_v, rows_v, zbuf, acc_sh, sem,
       ones_v, degst, deg_sh) = refs
    else:
      (agg_out, src_v, dst_v, rows_v, zbuf, acc_sh, sem) = refs
    cid = lax.axis_index("c")
    sid = lax.axis_index("s")
    wid = sid * NC + cid
    z16 = jnp.zeros((16,), jnp.float32)

    def zrow(i, carry):
      for k in range(8):
        zbuf[i, pl.ds(k * 16, 16)] = z16
      return carry
    lax.fori_loop(0, C, zrow, 0)
    row0 = sid * RPT
    for j in range(RPT // C):
      pltpu.sync_copy(zbuf, acc_sh.at[pl.ds(row0 + j * C, C)])
    if with_deg:
      def onesrow(i, carry):
        ones_v[i] = jnp.full((16,), 1.0, jnp.float32)
        return carry
      lax.fori_loop(0, C, onesrow, 0)
      def zdrow(i, carry):
        degst[i] = z16
        return carry
      lax.fori_loop(0, RPT, zdrow, 0)
      pltpu.sync_copy(degst, deg_sh.at[pl.ds(row0, RPT)])
    plsc.subcore_barrier()

    ebase = wid * EPW
    def chunk(i, carry):
      base = ebase + i * C
      pltpu.sync_copy(src_hbm.at[pl.ds(base, C)], src_v)
      pltpu.sync_copy(dst_hbm.at[pl.ds(base, C)], dst_v)
      pltpu.async_copy(x_hbm.at[src_v], rows_v, sem).wait()
      pltpu.sync_copy(rows_v, acc_sh.at[dst_v], add=True)
      if with_deg:
        pltpu.sync_copy(ones_v, deg_sh.at[dst_v], add=True)
      return carry
    lax.fori_loop(0, NCH, chunk, 0)
    plsc.subcore_barrier()

    obase = cid * NPAD + row0
    for j in range(RPT // C):
      pltpu.sync_copy(acc_sh.at[pl.ds(row0 + j * C, C)], zbuf)
      pltpu.sync_copy(zbuf, agg_out.at[pl.ds(obase + j * C, C)])
    if with_deg:
      pltpu.sync_copy(deg_sh.at[pl.ds(row0, RPT)], degst)
      pltpu.sync_copy(degst, deg_out.at[pl.ds(obase, RPT)])

  return pl.kernel(
      body,
      out_type=out_types if with_deg else out_types[0],
      mesh=_mesh,
      scratch_types=scratch,
      compiler_params=pltpu.CompilerParams(use_tc_tiling_on_sc=False),
  )


_edge_agg_deg = _make_edge_agg(True)
_edge_agg = _make_edge_agg(False)


def _gather_body(h_hbm, ids_hbm, out_hbm, idx_v, rows_v, sem):
  wid = lax.axis_index("s") * NC + lax.axis_index("c")
  base = wid * IPW
  for j in range(IPW // GC):
    pltpu.sync_copy(ids_hbm.at[pl.ds(base + j * GC, GC)], idx_v)
    pltpu.async_copy(h_hbm.at[idx_v], rows_v, sem).wait()
    pltpu.sync_copy(rows_v, out_hbm.at[pl.ds(base + j * GC, GC)])


_gather_rows = pl.kernel(
    _gather_body,
    out_type=jax.ShapeDtypeStruct((IDS, 128), jnp.float32),
    mesh=_mesh,
    scratch_types=[
        pltpu.VMEM((GC,), jnp.int32),
        pltpu.VMEM((GC, 128), jnp.float32),
        pltpu.SemaphoreType.DMA,
    ],
    compiler_params=pltpu.CompilerParams(use_tc_tiling_on_sc=False),
)


RB = 1280


def _layer_kernel(aggp, degp, nf, wn0, wr0, b0, wn1, wr1, b1, y_ref, xr_ref):
  agg = aggp[0] + aggp[1]
  deg = degp[0, :, 0:1] + degp[1, :, 0:1]
  rd = 1.0 / jnp.maximum(deg, 1.0)
  mean0 = agg * rd
  h = jnp.dot(mean0, wn0[...], preferred_element_type=jnp.float32)
  h = h + jnp.dot(nf[...], wr0[...], preferred_element_type=jnp.float32)
  h = jnp.maximum(h + b0[...], 0.0)
  y_ref[...] = jnp.dot(h, wn1[...], preferred_element_type=jnp.float32)
  xr_ref[...] = jnp.dot(h, wr1[...], preferred_element_type=jnp.float32) + b1[...]


def _layer_call(aggp, degp, nf, wn0, wr0, b0, wn1, wr1, b1):
  return pl.pallas_call(
      _layer_kernel,
      grid=(NPAD // RB,),
      in_specs=[
          pl.BlockSpec((NC, RB, 128), lambda i: (0, i, 0)),
          pl.BlockSpec((NC, RB, 16), lambda i: (0, i, 0)),
          pl.BlockSpec((RB, 128), lambda i: (i, 0)),
          pl.BlockSpec((D_IN, D_H), lambda i: (0, 0)),
          pl.BlockSpec((D_IN, D_H), lambda i: (0, 0)),
          pl.BlockSpec((1, D_H), lambda i: (0, 0)),
          pl.BlockSpec((D_H, D_OUT), lambda i: (0, 0)),
          pl.BlockSpec((D_H, D_OUT), lambda i: (0, 0)),
          pl.BlockSpec((1, D_OUT), lambda i: (0, 0)),
      ],
      out_specs=[
          pl.BlockSpec((RB, 128), lambda i: (i, 0)),
          pl.BlockSpec((RB, 128), lambda i: (i, 0)),
      ],
      out_shape=[
          jax.ShapeDtypeStruct((NPAD, 128), jnp.float32),
          jax.ShapeDtypeStruct((NPAD, 128), jnp.float32),
      ],
  )(aggp, degp, nf, wn0, wr0, b0, wn1, wr1, b1)


def _h1_kernel(aggp, degp, xr, out_ref):
  agg = aggp[0] + aggp[1]
  deg = degp[0, :, 0:1] + degp[1, :, 0:1]
  out_ref[...] = agg * (1.0 / jnp.maximum(deg, 1.0)) + xr[...]


def _h1_call(aggp, degp, xr):
  return pl.pallas_call(
      _h1_kernel,
      grid=(NPAD // RB,),
      in_specs=[
          pl.BlockSpec((NC, RB, 128), lambda i: (0, i, 0)),
          pl.BlockSpec((NC, RB, 16), lambda i: (0, i, 0)),
          pl.BlockSpec((RB, 128), lambda i: (i, 0)),
      ],
      out_specs=pl.BlockSpec((RB, 128), lambda i: (i, 0)),
      out_shape=jax.ShapeDtypeStruct((NPAD, 128), jnp.float32),
  )(aggp, degp, xr)


def _score_kernel(f, wp, pos_ref, neg_ref):
  s = f[0:BATCH]
  p = f[BATCH:2 * BATCH]
  n = f[2 * BATCH:3 * BATCH]
  w = wp[...]
  pos_ref[...] = jnp.sum(s * p * w, axis=1, keepdims=True)
  neg_ref[...] = jnp.sum(s * n * w, axis=1, keepdims=True)


def _score_call(feats, wp):
  return pl.pallas_call(
      _score_kernel,
      out_shape=[
          jax.ShapeDtypeStruct((BATCH, 1), jnp.float32),
          jax.ShapeDtypeStruct((BATCH, 1), jnp.float32),
      ],
  )(feats, wp)


def kernel(src_ids, pos_dst_ids, neg_dst_ids, node_feat, edge_index,
           Wn0, Wr0, b0, Wn1, Wr1, b1, w_pred):
  f32 = jnp.float32
  nf_pad = jnp.concatenate(
      [node_feat.astype(f32), jnp.zeros((NPAD - N, D_IN), f32)], axis=0)
  src = edge_index[0].astype(jnp.int32)
  dst = edge_index[1].astype(jnp.int32)
  ids = jnp.concatenate([src_ids, pos_dst_ids, neg_dst_ids]).astype(jnp.int32)

  aggp0f, degpf = _edge_agg_deg(nf_pad, src, dst)
  aggp0 = aggp0f.reshape(NC, NPAD, 128)
  degp = degpf.reshape(NC, NPAD, 16)

  y, xr = _layer_call(aggp0, degp, nf_pad, Wn0, Wr0,
                      b0.reshape(1, -1), Wn1, Wr1, b1.reshape(1, -1))

  aggp1 = _edge_agg(y, src, dst).reshape(NC, NPAD, 128)
  h1 = _h1_call(aggp1, degp, xr)

  feats = _gather_rows(h1, ids)
  pos, neg = _score_call(feats, w_pred.reshape(1, -1))
  return (pos.reshape(-1), neg.reshape(-1))
